# baseline (speedup 1.0000x reference)
import jax
import jax.numpy as jnp
from jax import lax
from jax.experimental import pallas as pl
from jax.experimental.pallas import tpu as pltpu
from jax.experimental.pallas import tpu_sc as plsc

_K = 100
_CAP = 144
_BUF = 160
_NJV = _CAP // 16
_T25 = 0x3E800000
_N = 20000
_NV = _N // 16


def _dense_body(x_ref, key_ref, pay_ref):
    xb = x_ref[0]
    ncls = xb.shape[0] - 4
    n = xb.shape[1]
    cx = xb[0:1, :]
    cy = xb[1:2, :]
    hw = xb[2:3, :] * 0.5
    hh = xb[3:4, :] * 0.5
    scores = xb[4:, :]

    best = jnp.max(scores, axis=0, keepdims=True)
    ci = jax.lax.broadcasted_iota(jnp.int32, (ncls, n), 0)
    cls = jnp.min(jnp.where(scores == best, ci, ncls * 2),
                  axis=0, keepdims=True)

    bits = jax.lax.bitcast_convert_type(best, jnp.int32)
    key = jnp.where(bits >= 0, bits, bits ^ jnp.int32(0x7FFFFFFF))

    key_ref[0] = key
    z = jnp.zeros_like(best)
    pay_ref[0] = jnp.concatenate(
        [cls.astype(jnp.float32),
         cx - hw, cy - hh, cx + hw, cy + hh,
         best, z, z], axis=0)


def _sc_body(key_hbm, pay_hbm,
             scores_hbm, cls_hbm, boxes_hbm, meta_hbm,
             keys_v, key2_v, gid2_v, hist_v, hb_v, eb_v, cidx_v, ckey_v,
             osc_v, ocl_v, obox_v, met_v, pch_v):
    wid = lax.axis_index("s") * 2 + lax.axis_index("c")

    @pl.when(wid < 4)
    def _():
        b = wid
        koff = pl.multiple_of(b * _N, 16)
        pltpu.sync_copy(key_hbm.at[pl.ds(koff, _N)], keys_v)

        lane = lax.broadcasted_iota(jnp.int32, (16,), 0)
        laneb = lane * 256
        ones = jnp.ones((16,), jnp.int32)
        zv = jnp.zeros((16,), jnp.int32)

        def scan_hist(tot, kp):
            limit = tot - kp
            carry = jnp.int32(0)
            cntb = jnp.int32(0)
            for v in range(16):
                hv = hist_v[v * 16:(v + 1) * 16]
                for l in range(1, 16):
                    hv = hv + hist_v[l * 256 + v * 16:l * 256 + (v + 1) * 16]
                c = plsc.cumsum(hv)
                excl = jnp.full((16,), carry, jnp.int32) + c - hv
                hb_v[v * 16:(v + 1) * 16] = hv
                eb_v[v * 16:(v + 1) * 16] = excl
                cntb = cntb + jnp.sum((excl <= limit).astype(jnp.int32))
                carry = carry + jnp.sum(hv)
            bsel = cntb - 1
            bful = jnp.full((16,), bsel, jnp.int32)
            h_b = jnp.max(plsc.load_gather(hb_v, [bful]))
            e_b = jnp.max(plsc.load_gather(eb_v, [bful]))
            above = tot - (e_b + h_b)
            return bsel, h_b, above

        def zero_hist():
            for l in range(16):
                for v in range(16):
                    hist_v[l * 256 + v * 16:l * 256 + (v + 1) * 16] = zv

        zero_hist()

        def habody(i, c25v):
            for u in range(4):
                off = pl.multiple_of(i * 64 + u * 16, 16)
                kv = keys_v[pl.ds(off, 16)]
                bins = jnp.bitwise_xor(lax.shift_right_logical(kv, 24), 128)
                plsc.addupdate_scatter(hist_v, [laneb + bins], ones)
                c25v = c25v + (kv > _T25).astype(jnp.int32)
            return c25v

        c25v = lax.fori_loop(0, _NV // 4, habody, zv)
        for t in range((_NV // 4) * 16 * 4, _N, 16):
            kv = keys_v[t:t + 16]
            bins = jnp.bitwise_xor(lax.shift_right_logical(kv, 24), 128)
            plsc.addupdate_scatter(hist_v, [laneb + bins], ones)
            c25v = c25v + (kv > _T25).astype(jnp.int32)
        nd = jnp.minimum(jnp.sum(c25v), _K)

        bsel, h_b, above = scan_hist(jnp.int32(_N), jnp.int32(_K))
        kp = jnp.int32(_K) - above
        raw0 = jnp.bitwise_xor(bsel, 128)
        pref = raw0
        tot = h_b
        thr_a = lax.shift_left(raw0, 24)

        def sbody(i, offv):
            for u in range(4):
                off = pl.multiple_of(i * 64 + u * 16, 16)
                kv = keys_v[pl.ds(off, 16)]
                m = kv >= thr_a
                pos = offv + plsc.cumsum(m.astype(jnp.int32)) - 1
                plsc.store_scatter(key2_v, [pos], kv, mask=m)
                plsc.store_scatter(gid2_v, [pos], lane + (i * 64 + u * 16),
                                   mask=m)
                offv = offv + plsc.all_reduce_population_count(m)
            return offv

        offv = lax.fori_loop(0, _NV // 4, sbody, zv)
        for t in range((_NV // 4) * 16 * 4, _N, 16):
            kv = keys_v[t:t + 16]
            m = kv >= thr_a
            pos = offv + plsc.cumsum(m.astype(jnp.int32)) - 1
            plsc.store_scatter(key2_v, [pos], kv, mask=m)
            plsc.store_scatter(gid2_v, [pos], lane + t, mask=m)
            offv = offv + plsc.all_reduce_population_count(m)
        n2 = jnp.max(offv)
        nv2 = lax.shift_right_logical(n2 + 15, 4)
        n2spl = jnp.full((16,), n2, jnp.int32)

        for p in range(1, 4):
            zero_hist()
            shift = 24 - 8 * p

            def hbody(i, c, shift=shift, pref=pref):
                off = pl.multiple_of(i * 16, 16)
                kv = key2_v[pl.ds(off, 16)]
                bins = lax.shift_right_logical(kv, shift) & 255
                hi = lax.shift_right_logical(kv, shift + 8)
                okm = (hi == pref) & (lane + i * 16 < n2spl)
                plsc.addupdate_scatter(hist_v, [laneb + bins], ones,
                                       mask=okm)
                return c

            lax.fori_loop(0, nv2, hbody, jnp.int32(0))
            bsel, h_b, above = scan_hist(tot, kp)
            kp = kp - above
            pref = jnp.bitwise_or(lax.shift_left(pref, 8), bsel)
            tot = h_b
        thr = pref

        def cbody(i, offv):
            off = pl.multiple_of(i * 16, 16)
            kv = key2_v[pl.ds(off, 16)]
            m = (kv >= thr) & (lane + i * 16 < n2spl)
            pos = offv + plsc.cumsum(m.astype(jnp.int32)) - 1
            gv = gid2_v[pl.ds(off, 16)]
            plsc.store_scatter(cidx_v, [pos], gv, mask=m & (pos < _CAP))
            offv = offv + plsc.all_reduce_population_count(m)
            return offv

        offv = lax.fori_loop(0, nv2, cbody, zv)
        cnt = jnp.minimum(jnp.max(offv), _CAP)

        cspl = jnp.full((16,), cnt, jnp.int32)
        kjs = []
        ijs = []
        for jv in range(_NJV):
            slot = lane + jv * 16
            iv = cidx_v[jv * 16:(jv + 1) * 16]
            bad = slot >= cspl
            ivc = jnp.clip(iv, 0, _N - 1)
            kv = plsc.load_gather(keys_v, [ivc])
            kv = jnp.where(bad, jnp.int32(-2147483647 - 1), kv)
            ivc = jnp.where(bad, jnp.int32(1000000), ivc)
            ckey_v[jv * 16:(jv + 1) * 16] = kv
            cidx_v[jv * 16:(jv + 1) * 16] = ivc
            kjs.append(kv)
            ijs.append(ivc)

        def rbody(i, ranks):
            isp = jnp.full((16,), i, jnp.int32)
            ki = plsc.load_gather(ckey_v, [isp])
            ii = plsc.load_gather(cidx_v, [isp])
            out = []
            for jv in range(_NJV):
                beats = (ki > kjs[jv]) | ((ki == kjs[jv]) & (ii < ijs[jv]))
                out.append(ranks[jv] + beats.astype(jnp.int32))
            return tuple(out)

        ranks = lax.fori_loop(0, _CAP, rbody,
                              tuple(zv for _ in range(_NJV)))

        for jv in range(_NJV):
            rk = ranks[jv]
            okm = rk < _K
            sbits = jnp.where(kjs[jv] >= 0, kjs[jv],
                              kjs[jv] ^ jnp.int32(0x7FFFFFFF))
            plsc.store_scatter(osc_v, [rk], plsc.bitcast(sbits, jnp.float32),
                               mask=okm)

        for cc in range(5):
            poff = pl.multiple_of((b * 8 + cc) * _N, 16)
            pltpu.sync_copy(pay_hbm.at[pl.ds(poff, _N)], pch_v)
            for jv in range(_NJV):
                civ = jnp.clip(cidx_v[jv * 16:(jv + 1) * 16], 0, _N - 1)
                rk = ranks[jv]
                okm = rk < _K
                v = plsc.load_gather(pch_v, [civ])
                if cc == 0:
                    plsc.store_scatter(ocl_v, [rk], v.astype(jnp.int32),
                                       mask=okm)
                else:
                    plsc.store_scatter(obox_v, [(cc - 1) * _BUF + rk], v,
                                       mask=okm)

        met_v[...] = jnp.where(lane == 0, nd, jnp.where(lane == 1, cnt, 0))
        pltpu.sync_copy(met_v, meta_hbm.at[pl.ds(b * 16, 16)])
        pltpu.sync_copy(osc_v.at[pl.ds(0, 128)],
                        scores_hbm.at[pl.ds(b * 128, 128)])
        pltpu.sync_copy(ocl_v.at[pl.ds(0, 128)],
                        cls_hbm.at[pl.ds(b * 128, 128)])
        for cc in range(4):
            pltpu.sync_copy(obox_v.at[pl.ds(cc * _BUF, 128)],
                            boxes_hbm.at[pl.ds((b * 4 + cc) * 128, 128)])


def kernel(x):
    b, c, n = x.shape
    key, pay = pl.pallas_call(
        _dense_body,
        grid=(b,),
        in_specs=[pl.BlockSpec((1, c, n), lambda i: (i, 0, 0))],
        out_specs=[pl.BlockSpec((1, 1, n), lambda i: (i, 0, 0)),
                   pl.BlockSpec((1, 8, n), lambda i: (i, 0, 0))],
        out_shape=[jax.ShapeDtypeStruct((b, 1, n), jnp.int32),
                   jax.ShapeDtypeStruct((b, 8, n), jnp.float32)],
    )(x)
    key = jnp.reshape(key, (b * n,))
    pay = jnp.reshape(pay, (b * 8 * n,))

    sc = pl.kernel(
        _sc_body,
        out_type=[jax.ShapeDtypeStruct((b * 128,), jnp.float32),
                  jax.ShapeDtypeStruct((b * 128,), jnp.int32),
                  jax.ShapeDtypeStruct((b * 4 * 128,), jnp.float32),
                  jax.ShapeDtypeStruct((b * 16,), jnp.int32)],
        mesh=plsc.VectorSubcoreMesh(core_axis_name="c",
                                    subcore_axis_name="s"),
        compiler_params=pltpu.CompilerParams(needs_layout_passes=False),
        scratch_types=[
            pltpu.VMEM((_N,), jnp.int32),
            pltpu.VMEM((_N,), jnp.int32),
            pltpu.VMEM((_N,), jnp.int32),
            pltpu.VMEM((4096,), jnp.int32),
            pltpu.VMEM((256,), jnp.int32),
            pltpu.VMEM((256,), jnp.int32),
            pltpu.VMEM((_BUF,), jnp.int32),
            pltpu.VMEM((_BUF,), jnp.int32),
            pltpu.VMEM((_BUF,), jnp.float32),
            pltpu.VMEM((_BUF,), jnp.int32),
            pltpu.VMEM((4 * _BUF,), jnp.float32),
            pltpu.VMEM((16,), jnp.int32),
            pltpu.VMEM((_N,), jnp.float32),
        ],
    )
    scores_f, cls_f, boxes_f, meta_f = sc(key, pay)

    num_det = jnp.reshape(meta_f, (b, 16))[:, :1]
    boxes_t = jnp.reshape(boxes_f, (b, 4, 128))
    det_boxes = jnp.transpose(boxes_t[:, :, :_K], (0, 2, 1))
    det_scores = jnp.reshape(scores_f, (b, 128))[:, :_K]
    det_classes = jnp.reshape(cls_f, (b, 128))[:, :_K]
    return (num_det, det_boxes, det_scores, det_classes)


# speedup vs baseline: 1.0653x; 1.0653x over previous
import jax
import jax.numpy as jnp
from jax import lax
from jax.experimental import pallas as pl
from jax.experimental.pallas import tpu as pltpu
from jax.experimental.pallas import tpu_sc as plsc

_K = 100
_CAP = 144
_BUF = 160
_NJV = _CAP // 16
_T25 = 0x3E800000
_N = 20000
_NV = _N // 16


def _dense_body(x_ref, key_ref, pay_ref):
    xb = x_ref[0]
    ncls = xb.shape[0] - 4
    n = xb.shape[1]
    cx = xb[0:1, :]
    cy = xb[1:2, :]
    hw = xb[2:3, :] * 0.5
    hh = xb[3:4, :] * 0.5
    scores = xb[4:, :]

    best = jnp.max(scores, axis=0, keepdims=True)
    ci = jax.lax.broadcasted_iota(jnp.int32, (ncls, n), 0)
    cls = jnp.min(jnp.where(scores == best, ci, ncls * 2),
                  axis=0, keepdims=True)

    bits = jax.lax.bitcast_convert_type(best, jnp.int32)
    key = jnp.where(bits >= 0, bits, bits ^ jnp.int32(0x7FFFFFFF))

    key_ref[0] = key
    z = jnp.zeros_like(best)
    pay_ref[0] = jnp.concatenate(
        [cls.astype(jnp.float32),
         cx - hw, cy - hh, cx + hw, cy + hh,
         best, z, z], axis=0)


def _sc_body(key_hbm, pay_hbm,
             scores_hbm, cls_hbm, boxes_hbm, meta_hbm,
             keys_v, hist_v, hb_v, eb_v, cidx_v, ckey_v,
             osc_v, ocl_v, obox_v, met_v, pch_v):
    wid = lax.axis_index("s") * 2 + lax.axis_index("c")

    @pl.when(wid < 4)
    def _():
        b = wid
        koff = pl.multiple_of(b * _N, 16)
        pltpu.sync_copy(key_hbm.at[pl.ds(koff, _N)], keys_v)

        lane = lax.broadcasted_iota(jnp.int32, (16,), 0)
        laneb = lane * 256
        ones = jnp.ones((16,), jnp.int32)
        zv = jnp.zeros((16,), jnp.int32)

        pref = jnp.int32(0)
        kp = jnp.int32(_K)
        tot = jnp.int32(_N)
        for p in range(4):
            for l in range(16):
                for v in range(16):
                    hist_v[l * 256 + v * 16:l * 256 + (v + 1) * 16] = zv
            shift = 24 - 8 * p

            def hbody(i, c, p=p, shift=shift, pref=pref):
                off = pl.multiple_of(i * 16, 16)
                kv = keys_v[pl.ds(off, 16)]
                byte = lax.shift_right_logical(kv, shift)
                if p == 0:
                    bins = jnp.bitwise_xor(byte & 255, 128)
                    plsc.addupdate_scatter(hist_v, [laneb + bins], ones)
                else:
                    bins = byte & 255
                    hi = lax.shift_right_logical(kv, shift + 8)
                    plsc.addupdate_scatter(hist_v, [laneb + bins], ones,
                                           mask=hi == pref)
                return c

            lax.fori_loop(0, _NV, hbody, jnp.int32(0))

            limit = tot - kp
            carry = jnp.int32(0)
            cntb = jnp.int32(0)
            for v in range(16):
                hv = hist_v[v * 16:(v + 1) * 16]
                for l in range(1, 16):
                    hv = hv + hist_v[l * 256 + v * 16:l * 256 + (v + 1) * 16]
                c = plsc.cumsum(hv)
                excl = jnp.full((16,), carry, jnp.int32) + c - hv
                hb_v[v * 16:(v + 1) * 16] = hv
                eb_v[v * 16:(v + 1) * 16] = excl
                cntb = cntb + jnp.sum((excl <= limit).astype(jnp.int32))
                carry = carry + jnp.sum(hv)
            bsel = cntb - 1
            bful = jnp.full((16,), bsel, jnp.int32)
            h_b = jnp.max(plsc.load_gather(hb_v, [bful]))
            e_b = jnp.max(plsc.load_gather(eb_v, [bful]))
            above = tot - (e_b + h_b)
            kp = kp - above
            raw = jnp.bitwise_xor(bsel, 128) if p == 0 else bsel
            pref = jnp.bitwise_or(lax.shift_left(pref, 8), raw)
            tot = h_b
        thr = pref

        def cbody(i, carry):
            offv, c25v = carry
            off = pl.multiple_of(i * 16, 16)
            kv = keys_v[pl.ds(off, 16)]
            m = kv >= thr
            pos = offv + plsc.cumsum(m.astype(jnp.int32)) - 1
            gidx = lane + i * 16
            plsc.store_scatter(cidx_v, [pos], gidx,
                               mask=m & (pos < _CAP))
            offv = offv + plsc.all_reduce_population_count(m)
            c25v = c25v + (kv > _T25).astype(jnp.int32)
            return offv, c25v

        offv, c25v = lax.fori_loop(0, _NV, cbody, (zv, zv))
        cnt = jnp.minimum(jnp.max(offv), _CAP)
        nd = jnp.minimum(jnp.sum(c25v), _K)

        cspl = jnp.full((16,), cnt, jnp.int32)
        kjs = []
        ijs = []
        for jv in range(_NJV):
            slot = lane + jv * 16
            iv = cidx_v[jv * 16:(jv + 1) * 16]
            bad = slot >= cspl
            ivc = jnp.clip(iv, 0, _N - 1)
            kv = plsc.load_gather(keys_v, [ivc])
            kv = jnp.where(bad, jnp.int32(-2147483647 - 1), kv)
            ivc = jnp.where(bad, jnp.int32(1000000), ivc)
            ckey_v[jv * 16:(jv + 1) * 16] = kv
            cidx_v[jv * 16:(jv + 1) * 16] = ivc
            kjs.append(kv)
            ijs.append(ivc)

        def rbody(i, ranks):
            isp = jnp.full((16,), i, jnp.int32)
            ki = plsc.load_gather(ckey_v, [isp])
            ii = plsc.load_gather(cidx_v, [isp])
            out = []
            for jv in range(_NJV):
                beats = (ki > kjs[jv]) | ((ki == kjs[jv]) & (ii < ijs[jv]))
                out.append(ranks[jv] + beats.astype(jnp.int32))
            return tuple(out)

        ranks = lax.fori_loop(0, _CAP, rbody,
                              tuple(zv for _ in range(_NJV)))

        for jv in range(_NJV):
            rk = ranks[jv]
            okm = rk < _K
            sbits = jnp.where(kjs[jv] >= 0, kjs[jv],
                              kjs[jv] ^ jnp.int32(0x7FFFFFFF))
            plsc.store_scatter(osc_v, [rk], plsc.bitcast(sbits, jnp.float32),
                               mask=okm)

        for cc in range(5):
            poff = pl.multiple_of((b * 8 + cc) * _N, 16)
            pltpu.sync_copy(pay_hbm.at[pl.ds(poff, _N)], pch_v)
            for jv in range(_NJV):
                civ = jnp.clip(cidx_v[jv * 16:(jv + 1) * 16], 0, _N - 1)
                rk = ranks[jv]
                okm = rk < _K
                v = plsc.load_gather(pch_v, [civ])
                if cc == 0:
                    plsc.store_scatter(ocl_v, [rk], v.astype(jnp.int32),
                                       mask=okm)
                else:
                    plsc.store_scatter(obox_v, [(cc - 1) * _BUF + rk], v,
                                       mask=okm)

        met_v[...] = jnp.where(lane == 0, nd, jnp.where(lane == 1, cnt, 0))
        pltpu.sync_copy(met_v, meta_hbm.at[pl.ds(b * 16, 16)])
        pltpu.sync_copy(osc_v.at[pl.ds(0, 128)],
                        scores_hbm.at[pl.ds(b * 128, 128)])
        pltpu.sync_copy(ocl_v.at[pl.ds(0, 128)],
                        cls_hbm.at[pl.ds(b * 128, 128)])
        for cc in range(4):
            pltpu.sync_copy(obox_v.at[pl.ds(cc * _BUF, 128)],
                            boxes_hbm.at[pl.ds((b * 4 + cc) * 128, 128)])


def kernel(x):
    b, c, n = x.shape
    key, pay = pl.pallas_call(
        _dense_body,
        grid=(b,),
        in_specs=[pl.BlockSpec((1, c, n), lambda i: (i, 0, 0))],
        out_specs=[pl.BlockSpec((1, 1, n), lambda i: (i, 0, 0)),
                   pl.BlockSpec((1, 8, n), lambda i: (i, 0, 0))],
        out_shape=[jax.ShapeDtypeStruct((b, 1, n), jnp.int32),
                   jax.ShapeDtypeStruct((b, 8, n), jnp.float32)],
    )(x)
    key = jnp.reshape(key, (b * n,))
    pay = jnp.reshape(pay, (b * 8 * n,))

    sc = pl.kernel(
        _sc_body,
        out_type=[jax.ShapeDtypeStruct((b * 128,), jnp.float32),
                  jax.ShapeDtypeStruct((b * 128,), jnp.int32),
                  jax.ShapeDtypeStruct((b * 4 * 128,), jnp.float32),
                  jax.ShapeDtypeStruct((b * 16,), jnp.int32)],
        mesh=plsc.VectorSubcoreMesh(core_axis_name="c",
                                    subcore_axis_name="s"),
        compiler_params=pltpu.CompilerParams(needs_layout_passes=False),
        scratch_types=[
            pltpu.VMEM((_N,), jnp.int32),
            pltpu.VMEM((4096,), jnp.int32),
            pltpu.VMEM((256,), jnp.int32),
            pltpu.VMEM((256,), jnp.int32),
            pltpu.VMEM((_BUF,), jnp.int32),
            pltpu.VMEM((_BUF,), jnp.int32),
            pltpu.VMEM((_BUF,), jnp.float32),
            pltpu.VMEM((_BUF,), jnp.int32),
            pltpu.VMEM((4 * _BUF,), jnp.float32),
            pltpu.VMEM((16,), jnp.int32),
            pltpu.VMEM((_N,), jnp.float32),
        ],
    )
    scores_f, cls_f, boxes_f, meta_f = sc(key, pay)

    num_det = jnp.reshape(meta_f, (b, 16))[:, :1]
    boxes_t = jnp.reshape(boxes_f, (b, 4, 128))
    det_boxes = jnp.transpose(boxes_t[:, :, :_K], (0, 2, 1))
    det_scores = jnp.reshape(scores_f, (b, 128))[:, :_K]
    det_classes = jnp.reshape(cls_f, (b, 128))[:, :_K]
    return (num_det, det_boxes, det_scores, det_classes)


# speedup vs baseline: 1.1161x; 1.0476x over previous
import jax
import jax.numpy as jnp
from jax import lax
from jax.experimental import pallas as pl
from jax.experimental.pallas import tpu as pltpu
from jax.experimental.pallas import tpu_sc as plsc

_K = 100
_CAP = 144
_BUF = 160
_NJV = _CAP // 16
_T25 = 0x3E800000
_N = 20000
_NV = _N // 16


def _dense_body(x_ref, key_ref, pay_ref):
    xb = x_ref[0]
    ncls = xb.shape[0] - 4
    n = xb.shape[1]
    cx = xb[0:1, :]
    cy = xb[1:2, :]
    hw = xb[2:3, :] * 0.5
    hh = xb[3:4, :] * 0.5
    scores = xb[4:, :]

    best = jnp.max(scores, axis=0, keepdims=True)
    ci = jax.lax.broadcasted_iota(jnp.int32, (ncls, n), 0)
    cls = jnp.min(jnp.where(scores == best, ci, ncls * 2),
                  axis=0, keepdims=True)

    bits = jax.lax.bitcast_convert_type(best, jnp.int32)
    key = jnp.where(bits >= 0, bits, bits ^ jnp.int32(0x7FFFFFFF))

    key_ref[0] = key
    z = jnp.zeros_like(best)
    pay_ref[0] = jnp.concatenate(
        [cls.astype(jnp.float32),
         cx - hw, cy - hh, cx + hw, cy + hh,
         best, z, z], axis=0)


def _sc_body(key_hbm, pay_hbm,
             scores_hbm, cls_hbm, boxes_hbm, meta_hbm,
             keys_v, hist_v, hb_v, eb_v, cidx_v, ckey_v,
             osc_v, ocl_v, obox_v, met_v, pch_v, psem):
    wid = lax.axis_index("s") * 2 + lax.axis_index("c")

    @pl.when(wid < 4)
    def _():
        b = wid
        koff = pl.multiple_of(b * _N, 16)
        pltpu.sync_copy(key_hbm.at[pl.ds(koff, _N)], keys_v)
        poff = pl.multiple_of(b * 8 * _N, 16)
        pcp = pltpu.async_copy(pay_hbm.at[pl.ds(poff, 5 * _N)], pch_v, psem)

        lane = lax.broadcasted_iota(jnp.int32, (16,), 0)
        laneb = lane * 256
        ones = jnp.ones((16,), jnp.int32)
        zv = jnp.zeros((16,), jnp.int32)

        pref = jnp.int32(0)
        kp = jnp.int32(_K)
        tot = jnp.int32(_N)
        for p in range(4):
            for l in range(16):
                for v in range(16):
                    hist_v[l * 256 + v * 16:l * 256 + (v + 1) * 16] = zv
            shift = 24 - 8 * p

            def hbody(i, c, p=p, shift=shift, pref=pref):
                off = pl.multiple_of(i * 16, 16)
                kv = keys_v[pl.ds(off, 16)]
                byte = lax.shift_right_logical(kv, shift)
                if p == 0:
                    bins = jnp.bitwise_xor(byte & 255, 128)
                    plsc.addupdate_scatter(hist_v, [laneb + bins], ones)
                else:
                    bins = byte & 255
                    hi = lax.shift_right_logical(kv, shift + 8)
                    plsc.addupdate_scatter(hist_v, [laneb + bins], ones,
                                           mask=hi == pref)
                return c

            lax.fori_loop(0, _NV, hbody, jnp.int32(0))

            limit = tot - kp
            carry = jnp.int32(0)
            cntb = jnp.int32(0)
            for v in range(16):
                hv = hist_v[v * 16:(v + 1) * 16]
                for l in range(1, 16):
                    hv = hv + hist_v[l * 256 + v * 16:l * 256 + (v + 1) * 16]
                c = plsc.cumsum(hv)
                excl = jnp.full((16,), carry, jnp.int32) + c - hv
                hb_v[v * 16:(v + 1) * 16] = hv
                eb_v[v * 16:(v + 1) * 16] = excl
                cntb = cntb + jnp.sum((excl <= limit).astype(jnp.int32))
                carry = carry + jnp.sum(hv)
            bsel = cntb - 1
            bful = jnp.full((16,), bsel, jnp.int32)
            h_b = jnp.max(plsc.load_gather(hb_v, [bful]))
            e_b = jnp.max(plsc.load_gather(eb_v, [bful]))
            above = tot - (e_b + h_b)
            kp = kp - above
            raw = jnp.bitwise_xor(bsel, 128) if p == 0 else bsel
            pref = jnp.bitwise_or(lax.shift_left(pref, 8), raw)
            tot = h_b
        thr = pref

        def cbody(i, carry):
            offv, c25v = carry
            off = pl.multiple_of(i * 16, 16)
            kv = keys_v[pl.ds(off, 16)]
            m = kv >= thr
            pos = offv + plsc.cumsum(m.astype(jnp.int32)) - 1
            gidx = lane + i * 16
            plsc.store_scatter(cidx_v, [pos], gidx,
                               mask=m & (pos < _CAP))
            offv = offv + plsc.all_reduce_population_count(m)
            c25v = c25v + (kv > _T25).astype(jnp.int32)
            return offv, c25v

        offv, c25v = lax.fori_loop(0, _NV, cbody, (zv, zv))
        cnt = jnp.minimum(jnp.max(offv), _CAP)
        nd = jnp.minimum(jnp.sum(c25v), _K)

        cspl = jnp.full((16,), cnt, jnp.int32)
        kjs = []
        ijs = []
        for jv in range(_NJV):
            slot = lane + jv * 16
            iv = cidx_v[jv * 16:(jv + 1) * 16]
            bad = slot >= cspl
            ivc = jnp.clip(iv, 0, _N - 1)
            kv = plsc.load_gather(keys_v, [ivc])
            kv = jnp.where(bad, jnp.int32(-2147483647 - 1), kv)
            ivc = jnp.where(bad, jnp.int32(1000000), ivc)
            ckey_v[jv * 16:(jv + 1) * 16] = kv
            cidx_v[jv * 16:(jv + 1) * 16] = ivc
            kjs.append(kv)
            ijs.append(ivc)

        def rbody(i, ranks):
            isp = jnp.full((16,), i, jnp.int32)
            ki = plsc.load_gather(ckey_v, [isp])
            ii = plsc.load_gather(cidx_v, [isp])
            out = []
            for jv in range(_NJV):
                beats = (ki > kjs[jv]) | ((ki == kjs[jv]) & (ii < ijs[jv]))
                out.append(ranks[jv] + beats.astype(jnp.int32))
            return tuple(out)

        ranks = lax.fori_loop(0, _CAP, rbody,
                              tuple(zv for _ in range(_NJV)))

        for jv in range(_NJV):
            rk = ranks[jv]
            okm = rk < _K
            sbits = jnp.where(kjs[jv] >= 0, kjs[jv],
                              kjs[jv] ^ jnp.int32(0x7FFFFFFF))
            plsc.store_scatter(osc_v, [rk], plsc.bitcast(sbits, jnp.float32),
                               mask=okm)

        pcp.wait()
        for cc in range(5):
            for jv in range(_NJV):
                civ = jnp.clip(cidx_v[jv * 16:(jv + 1) * 16], 0, _N - 1)
                rk = ranks[jv]
                okm = rk < _K
                v = plsc.load_gather(pch_v, [civ + cc * _N])
                if cc == 0:
                    plsc.store_scatter(ocl_v, [rk], v.astype(jnp.int32),
                                       mask=okm)
                else:
                    plsc.store_scatter(obox_v, [(cc - 1) * _BUF + rk], v,
                                       mask=okm)

        met_v[...] = jnp.where(lane == 0, nd, jnp.where(lane == 1, cnt, 0))
        pltpu.sync_copy(met_v, meta_hbm.at[pl.ds(b * 16, 16)])
        pltpu.sync_copy(osc_v.at[pl.ds(0, 128)],
                        scores_hbm.at[pl.ds(b * 128, 128)])
        pltpu.sync_copy(ocl_v.at[pl.ds(0, 128)],
                        cls_hbm.at[pl.ds(b * 128, 128)])
        for cc in range(4):
            pltpu.sync_copy(obox_v.at[pl.ds(cc * _BUF, 128)],
                            boxes_hbm.at[pl.ds((b * 4 + cc) * 128, 128)])


def kernel(x):
    b, c, n = x.shape
    key, pay = pl.pallas_call(
        _dense_body,
        grid=(b,),
        in_specs=[pl.BlockSpec((1, c, n), lambda i: (i, 0, 0))],
        out_specs=[pl.BlockSpec((1, 1, n), lambda i: (i, 0, 0)),
                   pl.BlockSpec((1, 8, n), lambda i: (i, 0, 0))],
        out_shape=[jax.ShapeDtypeStruct((b, 1, n), jnp.int32),
                   jax.ShapeDtypeStruct((b, 8, n), jnp.float32)],
    )(x)
    key = jnp.reshape(key, (b * n,))
    pay = jnp.reshape(pay, (b * 8 * n,))

    sc = pl.kernel(
        _sc_body,
        out_type=[jax.ShapeDtypeStruct((b * 128,), jnp.float32),
                  jax.ShapeDtypeStruct((b * 128,), jnp.int32),
                  jax.ShapeDtypeStruct((b * 4 * 128,), jnp.float32),
                  jax.ShapeDtypeStruct((b * 16,), jnp.int32)],
        mesh=plsc.VectorSubcoreMesh(core_axis_name="c",
                                    subcore_axis_name="s"),
        compiler_params=pltpu.CompilerParams(needs_layout_passes=False),
        scratch_types=[
            pltpu.VMEM((_N,), jnp.int32),
            pltpu.VMEM((4096,), jnp.int32),
            pltpu.VMEM((256,), jnp.int32),
            pltpu.VMEM((256,), jnp.int32),
            pltpu.VMEM((_BUF,), jnp.int32),
            pltpu.VMEM((_BUF,), jnp.int32),
            pltpu.VMEM((_BUF,), jnp.float32),
            pltpu.VMEM((_BUF,), jnp.int32),
            pltpu.VMEM((4 * _BUF,), jnp.float32),
            pltpu.VMEM((16,), jnp.int32),
            pltpu.VMEM((5 * _N,), jnp.float32),
            pltpu.SemaphoreType.DMA,
        ],
    )
    scores_f, cls_f, boxes_f, meta_f = sc(key, pay)

    num_det = jnp.reshape(meta_f, (b, 16))[:, :1]
    boxes_t = jnp.reshape(boxes_f, (b, 4, 128))
    det_boxes = jnp.transpose(boxes_t[:, :, :_K], (0, 2, 1))
    det_scores = jnp.reshape(scores_f, (b, 128))[:, :_K]
    det_classes = jnp.reshape(cls_f, (b, 128))[:, :_K]
    return (num_det, det_boxes, det_scores, det_classes)
